# baseline (speedup 1.0000x reference)
import functools

import jax
import jax.numpy as jnp
from jax import lax
from jax.experimental import pallas as pl
from jax.experimental.pallas import tpu as pltpu
from jax.experimental.pallas import tpu_sc as plsc


def _make_sc_kernel(B, K, V, P, NC, NS):
    NW = NC * NS
    BPW = B // NW
    C = 64
    NCHUNK = BPW // C
    J = C * K
    GSUB = 112
    NSUB = J // GSUB
    NG = C // 16

    mesh = plsc.VectorSubcoreMesh(core_axis_name="c", subcore_axis_name="s")

    @functools.partial(
        pl.kernel,
        out_type=jax.ShapeDtypeStruct((B * K,), jnp.float32),
        mesh=mesh,
        scratch_types=[
            pltpu.VMEM((C,), jnp.int32),
            pltpu.VMEM((NSUB, GSUB), jnp.int32),
            pltpu.VMEM((C, P), jnp.float32),
            pltpu.VMEM((J, P), jnp.float32),
            pltpu.VMEM((J,), jnp.float32),
            pltpu.SemaphoreType.DMA,
        ],
        compiler_params=pltpu.CompilerParams(
            needs_layout_passes=False, use_tc_tiling_on_sc=False),
    )
    def sc_kernel(x_hbm, s_hbm, emb_hbm, nemb_hbm, out_hbm,
                  xidx, sidx, erows, nrows, outv, sem):
        wid = lax.axis_index("s") * NC + lax.axis_index("c")
        iota16 = lax.iota(jnp.int32, 16)
        iotaK = iota16 * K

        def chunk_body(ci, _):
            nbase = wid * BPW + ci * C
            jbase = nbase * K

            pltpu.sync_copy(x_hbm.at[pl.ds(nbase, C)], xidx)
            descs = [pltpu.async_copy(emb_hbm.at[xidx], erows, sem)]
            for s in range(NSUB):
                pltpu.sync_copy(s_hbm.at[pl.ds(jbase + s * GSUB, GSUB)],
                                sidx.at[s])
                descs.append(
                    pltpu.async_copy(nemb_hbm.at[sidx.at[s]],
                                     nrows.at[pl.ds(s * GSUB, GSUB)], sem))
            for d in descs:
                d.wait()

            def group_body(g, _):
                nloc = g * 16 + iota16
                ev = [plsc.load_gather(erows, [nloc, jnp.full((16,), p, jnp.int32)])
                      for p in range(P)]
                for k in range(K):
                    row16 = iotaK + (g * (16 * K) + k)
                    acc = ev[0] * plsc.load_gather(
                        nrows, [row16, jnp.zeros((16,), jnp.int32)])
                    for p in range(1, P):
                        acc = acc + ev[p] * plsc.load_gather(
                            nrows, [row16, jnp.full((16,), p, jnp.int32)])
                    plsc.store_scatter(outv, [row16], acc)
                return 0

            lax.fori_loop(0, NG, group_body, 0)
            pltpu.sync_copy(outv, out_hbm.at[pl.ds(jbase, J)])
            return 0

        lax.fori_loop(0, NCHUNK, chunk_body, 0)

    return sc_kernel


def kernel(x, sampled, Emb, NEmb):
    B = x.shape[0]
    K = sampled.shape[1]
    V, P = Emb.shape
    try:
        info = plsc.get_sparse_core_info()
        NC, NS = info.num_cores, info.num_subcores
    except Exception:
        NC, NS = 2, 16
    fn = _make_sc_kernel(B, K, V, P, NC, NS)
    out = fn(x, sampled.reshape(-1), Emb, NEmb)
    return out.reshape(B, K)


# speedup vs baseline: 1.0257x; 1.0257x over previous
import functools

import jax
import jax.numpy as jnp
from jax import lax
from jax.experimental import pallas as pl
from jax.experimental.pallas import tpu as pltpu
from jax.experimental.pallas import tpu_sc as plsc


def _make_sc_kernel(B, K, V, P, NC, NS):
    NW = NC * NS
    BPW = B // NW
    C = 64
    NCHUNK = BPW // C
    J = C * K
    GSUB = 112
    NSUB = J // GSUB
    NG = C // 16
    NFLAT = J // 16
    MAGIC = (1 << 16) // K + 1

    mesh = plsc.VectorSubcoreMesh(core_axis_name="c", subcore_axis_name="s")

    @functools.partial(
        pl.kernel,
        out_type=jax.ShapeDtypeStruct((B, K), jnp.float32),
        mesh=mesh,
        scratch_types=[
            pltpu.VMEM((C,), jnp.int32),
            pltpu.VMEM((C, K), jnp.int32),
            pltpu.VMEM((J,), jnp.int32),
            pltpu.VMEM((C, P), jnp.float32),
            pltpu.VMEM((J, P), jnp.float32),
            pltpu.VMEM((C, K), jnp.float32),
            pltpu.SemaphoreType.DMA,
        ],
        compiler_params=pltpu.CompilerParams(
            needs_layout_passes=False, use_tc_tiling_on_sc=False),
    )
    def sc_kernel(x_hbm, s_hbm, emb_hbm, nemb_hbm, out_hbm,
                  xidx, sblk, sflat, erows, nrows, outv, sem):
        wid = lax.axis_index("s") * NC + lax.axis_index("c")
        iota16 = lax.iota(jnp.int32, 16)

        def chunk_body(ci, _):
            nbase = wid * BPW + ci * C

            pltpu.sync_copy(x_hbm.at[pl.ds(nbase, C)], xidx)
            pltpu.sync_copy(s_hbm.at[pl.ds(nbase, C)], sblk)

            def flat_body(g, _):
                j16 = g * 16 + iota16
                q = lax.shift_right_logical(j16 * MAGIC, 16)
                r = j16 - q * K
                sflat[pl.ds(g * 16, 16)] = plsc.load_gather(sblk, [q, r])
                return 0

            lax.fori_loop(0, NFLAT, flat_body, 0)

            descs = [pltpu.async_copy(emb_hbm.at[xidx], erows, sem)]
            for s in range(NSUB):
                descs.append(
                    pltpu.async_copy(nemb_hbm.at[sflat.at[pl.ds(s * GSUB, GSUB)]],
                                     nrows.at[pl.ds(s * GSUB, GSUB)], sem))
            for d in descs:
                d.wait()

            def group_body(g, _):
                nloc = g * 16 + iota16
                ev = [plsc.load_gather(erows, [nloc, jnp.full((16,), p, jnp.int32)])
                      for p in range(P)]
                for k in range(K):
                    row16 = (g * 16 + iota16) * K + k
                    acc = ev[0] * plsc.load_gather(
                        nrows, [row16, jnp.zeros((16,), jnp.int32)])
                    for p in range(1, P):
                        acc = acc + ev[p] * plsc.load_gather(
                            nrows, [row16, jnp.full((16,), p, jnp.int32)])
                    plsc.store_scatter(outv, [nloc, jnp.full((16,), k, jnp.int32)], acc)
                return 0

            lax.fori_loop(0, NG, group_body, 0)
            pltpu.sync_copy(outv, out_hbm.at[pl.ds(nbase, C)])
            return 0

        lax.fori_loop(0, NCHUNK, chunk_body, 0)

    return sc_kernel


def kernel(x, sampled, Emb, NEmb):
    B = x.shape[0]
    K = sampled.shape[1]
    V, P = Emb.shape
    try:
        info = plsc.get_sparse_core_info()
        NC, NS = info.num_cores, info.num_subcores
    except Exception:
        NC, NS = 2, 16
    fn = _make_sc_kernel(B, K, V, P, NC, NS)
    return fn(x, sampled, Emb, NEmb)
